# Initial kernel scaffold; baseline (speedup 1.0000x reference)
import jax
import jax.numpy as jnp
from jax.experimental import pallas as pl


def kernel(z, x, tc_w, tc_b, tl_w, tl_b, sc_w, sc_b, sl_w, sl_b, adj_w, adj_b):
    raise NotImplementedError("write your pallas kernel here")


# speedup vs baseline: 5.9500x; 5.9500x over previous
import jax
import jax.numpy as jnp
from jax.experimental import pallas as pl
from jax.experimental.pallas import tpu as pltpu

_PARALLEL = pltpu.CompilerParams(dimension_semantics=("parallel",))


def _conv_kernel(zp_ref, xp_ref, wt_ref, bt_ref, ws_ref, bs_ref, t_ref, s_ref):
    acc_t = jnp.zeros((768, 64), jnp.float32)
    acc_s = jnp.zeros((256, 536), jnp.float32)
    for i in range(3):
        for j in range(3):
            tap = i * 3 + j
            acc_t += jnp.dot(wt_ref[tap], zp_ref[:, i * 8 + j:i * 8 + j + 64],
                             preferred_element_type=jnp.float32)
            acc_s += jnp.dot(ws_ref[tap], xp_ref[:, i * 24 + j:i * 24 + j + 536],
                             preferred_element_type=jnp.float32)
    t_ref[...] = acc_t + bt_ref[...]
    s_ref[...] = acc_s + bs_ref[...]


def _xcorr_kernel(k_ref, s_ref, b_ref, out_ref):
    acc = jnp.broadcast_to(b_ref[...], (8, 408)).astype(jnp.float32)
    for di in range(6):
        for dj in range(6):
            off = di * 24 + dj
            acc = acc + jnp.dot(k_ref[di * 6 + dj], s_ref[:, off:off + 408],
                                preferred_element_type=jnp.float32)
    out_ref[...] = acc


def kernel(z, x, tc_w, tc_b, tl_w, tl_b, sc_w, sc_b, sl_w, sl_b, adj_w, adj_b):
    f32 = jnp.float32
    B, C = z.shape[0], z.shape[1]

    zp = jnp.zeros((B, C, 96), f32).at[:, :, :64].set(
        z.astype(f32).reshape(B, C, 64))
    xp = jnp.zeros((B, C, 640), f32).at[:, :, :576].set(
        x.astype(f32).reshape(B, C, 576))
    wt = jnp.concatenate([tc_w, tl_w], 0).astype(f32)
    wt9 = wt.transpose(2, 3, 0, 1).reshape(9, 768, C)
    ws = jnp.concatenate([sc_w, sl_w], 0).astype(f32)
    ws9 = ws.transpose(2, 3, 0, 1).reshape(9, 256, C)
    bt = jnp.concatenate([tc_b, tl_b]).astype(f32).reshape(768, 1)
    bs = jnp.concatenate([sc_b, sl_b]).astype(f32).reshape(256, 1)

    t_out, s_out = pl.pallas_call(
        _conv_kernel,
        out_shape=(jax.ShapeDtypeStruct((B, 768, 64), f32),
                   jax.ShapeDtypeStruct((B, 256, 536), f32)),
        grid=(B,),
        in_specs=[
            pl.BlockSpec((None, C, 96), lambda b: (b, 0, 0)),
            pl.BlockSpec((None, C, 640), lambda b: (b, 0, 0)),
            pl.BlockSpec((9, 768, C), lambda b: (0, 0, 0)),
            pl.BlockSpec((768, 1), lambda b: (0, 0)),
            pl.BlockSpec((9, 256, C), lambda b: (0, 0, 0)),
            pl.BlockSpec((256, 1), lambda b: (0, 0)),
        ],
        out_specs=(pl.BlockSpec((None, 768, 64), lambda b: (b, 0, 0)),
                   pl.BlockSpec((None, 256, 536), lambda b: (b, 0, 0))),
        compiler_params=_PARALLEL,
    )(zp, xp, wt9, bt, ws9, bs)

    tt = t_out.reshape(B, 6, C, 8, 8)[:, :, :, :6, :6]
    k_all = tt.transpose(0, 3, 4, 1, 2).reshape(B, 36, 6, C)
    aw = adj_w[:, :, 0, 0].astype(f32)
    klocp = jnp.einsum('pn,btnc->btpc', aw, k_all[:, :, 2:6, :])
    K = jnp.zeros((B, 36, 8, 2 * C), f32)
    K = K.at[:, :, 0:2, 0:C].set(k_all[:, :, 0:2, :])
    K = K.at[:, :, 2:6, C:2 * C].set(klocp)
    bias8 = jnp.zeros((8, 1), f32).at[2:6, 0].set(adj_b.astype(f32))

    out = pl.pallas_call(
        _xcorr_kernel,
        out_shape=jax.ShapeDtypeStruct((B, 8, 408), f32),
        grid=(B,),
        in_specs=[
            pl.BlockSpec((None, 36, 8, 2 * C), lambda b: (b, 0, 0, 0)),
            pl.BlockSpec((None, 256, 536), lambda b: (b, 0, 0)),
            pl.BlockSpec((8, 1), lambda b: (0, 0)),
        ],
        out_specs=pl.BlockSpec((None, 8, 408), lambda b: (b, 0, 0)),
        compiler_params=_PARALLEL,
    )(K, s_out, bias8)

    r = out.reshape(B, 8, 17, 24)[:, :, :, :17]
    return r[:, 0:2], r[:, 2:6]


# speedup vs baseline: 8.9929x; 1.5114x over previous
import jax
import jax.numpy as jnp
from jax.experimental import pallas as pl
from jax.experimental.pallas import tpu as pltpu

_PARALLEL = pltpu.CompilerParams(dimension_semantics=("parallel",))


def _conv_kernel(zt_ref, xf_ref, wtT_ref, btT_ref, ws_ref, bs_ref,
                 tT_ref, s_ref):
    acc_t = jnp.zeros((48, 768), jnp.float32)
    acc_s = jnp.zeros((256, 526), jnp.float32)
    for i in range(3):
        for j in range(3):
            tap = i * 3 + j
            zo = i * 8 + j
            xo = i * 24 + j
            acc_t += jnp.dot(zt_ref[zo:zo + 48, :], wtT_ref[tap],
                             preferred_element_type=jnp.float32)
            acc_s += jnp.dot(ws_ref[tap], xf_ref[:, xo:xo + 526],
                             preferred_element_type=jnp.float32)
    tT_ref[...] = acc_t + btT_ref[...]
    s_ref[...] = acc_s + bs_ref[...]


def _xcorr_kernel(k_ref, s_ref, ab_ref, out_ref):
    acc_c = jnp.zeros((2, 526), jnp.float32)
    acc_l = jnp.broadcast_to(ab_ref[...], (4, 526)).astype(jnp.float32)
    sc = s_ref[0:128, :]
    sl = s_ref[128:256, :]
    prev = None
    for di in range(5, -1, -1):
        for dj in range(5, -1, -1):
            off = di * 24 + dj
            if prev is not None:
                acc_c = jnp.roll(acc_c, off - prev, axis=1)
                acc_l = jnp.roll(acc_l, off - prev, axis=1)
            tap = di * 6 + dj
            acc_c = acc_c + jnp.dot(k_ref[tap, 0:2, :], sc,
                                    preferred_element_type=jnp.float32)
            acc_l = acc_l + jnp.dot(k_ref[tap, 2:6, :], sl,
                                    preferred_element_type=jnp.float32)
            prev = off
    out_ref[0:2, :] = acc_c[:, 0:401]
    out_ref[2:6, :] = acc_l[:, 0:401]


def kernel(z, x, tc_w, tc_b, tl_w, tl_b, sc_w, sc_b, sl_w, sl_b, adj_w, adj_b):
    f32 = jnp.float32
    B, C = z.shape[0], z.shape[1]

    zt = jnp.pad(z.astype(f32).reshape(B, C, 64).transpose(0, 2, 1),
                 ((0, 0), (0, 8), (0, 0)))
    xf = x.astype(f32).reshape(B, C, 576)
    aw = adj_w[:, :, 0, 0].astype(f32)
    wtl = jnp.einsum('pn,ncr->pcr', aw,
                     tl_w.astype(f32).reshape(4, C, C * 9)).reshape(4 * C, C * 9)
    wt_all = jnp.concatenate([tc_w.astype(f32).reshape(2 * C, C * 9), wtl], 0)
    wtT9 = wt_all.reshape(6 * C, C, 9).transpose(2, 1, 0)
    btl = (aw @ tl_b.astype(f32).reshape(4, C)).reshape(4 * C)
    btT = jnp.concatenate([tc_b.astype(f32), btl]).reshape(1, 6 * C)
    ws_all = jnp.concatenate([sc_w, sl_w], 0).astype(f32)
    ws9 = ws_all.transpose(2, 3, 0, 1).reshape(9, 2 * C, C)
    bs = jnp.concatenate([sc_b, sl_b]).astype(f32).reshape(2 * C, 1)

    tT, s_out = pl.pallas_call(
        _conv_kernel,
        out_shape=(jax.ShapeDtypeStruct((B, 48, 6 * C), f32),
                   jax.ShapeDtypeStruct((B, 2 * C, 526), f32)),
        grid=(B,),
        in_specs=[
            pl.BlockSpec((None, 72, C), lambda b: (b, 0, 0)),
            pl.BlockSpec((None, C, 576), lambda b: (b, 0, 0)),
            pl.BlockSpec((9, C, 6 * C), lambda b: (0, 0, 0)),
            pl.BlockSpec((1, 6 * C), lambda b: (0, 0)),
            pl.BlockSpec((9, 2 * C, C), lambda b: (0, 0, 0)),
            pl.BlockSpec((2 * C, 1), lambda b: (0, 0)),
        ],
        out_specs=(pl.BlockSpec((None, 48, 6 * C), lambda b: (b, 0, 0)),
                   pl.BlockSpec((None, 2 * C, 526), lambda b: (b, 0, 0))),
        compiler_params=_PARALLEL,
    )(zt, xf, wtT9, btT, ws9, bs)

    k_all = (tT.reshape(B, 6, 8, 6, C)[:, :, :6]
             .reshape(B, 36, 6, C))
    ab = adj_b.astype(f32).reshape(4, 1)

    out = pl.pallas_call(
        _xcorr_kernel,
        out_shape=jax.ShapeDtypeStruct((B, 8, 401), f32),
        grid=(B,),
        in_specs=[
            pl.BlockSpec((None, 36, 6, C), lambda b: (b, 0, 0, 0)),
            pl.BlockSpec((None, 2 * C, 526), lambda b: (b, 0, 0)),
            pl.BlockSpec((4, 1), lambda b: (0, 0)),
        ],
        out_specs=pl.BlockSpec((None, 8, 401), lambda b: (b, 0, 0)),
        compiler_params=_PARALLEL,
    )(k_all, s_out, ab)

    r = jnp.pad(out, ((0, 0), (0, 0), (0, 7))).reshape(B, 8, 17, 24)[:, :, :, :17]
    return r[:, 0:2], r[:, 2:6]


# speedup vs baseline: 12.1767x; 1.3540x over previous
import jax
import jax.numpy as jnp
from jax.experimental import pallas as pl
from jax.experimental.pallas import tpu as pltpu

_PARALLEL = pltpu.CompilerParams(dimension_semantics=("parallel",))
_GB = 8


def _conv_kernel(zt_ref, xf_ref, wtT_ref, btT_ref, ws_ref, bs_ref,
                 tT_ref, s_ref):
    for bb in range(_GB):
        acc_t = jnp.zeros((48, 768), jnp.float32)
        acc_s = jnp.zeros((256, 526), jnp.float32)
        for i in range(3):
            for j in range(3):
                tap = i * 3 + j
                zo = i * 8 + j
                xo = i * 24 + j
                acc_t += jnp.dot(zt_ref[bb, zo:zo + 48, :], wtT_ref[tap],
                                 preferred_element_type=jnp.float32)
                acc_s += jnp.dot(ws_ref[tap], xf_ref[bb, :, xo:xo + 526],
                                 preferred_element_type=jnp.float32)
        tT_ref[bb] = (acc_t + btT_ref[...]).astype(jnp.bfloat16)
        s_ref[bb] = (acc_s + bs_ref[...]).astype(jnp.bfloat16)


def _xcorr_kernel(k_ref, s_ref, ab_ref, out_ref):
    for bb in range(_GB):
        acc_c = jnp.zeros((2, 526), jnp.float32)
        acc_l = jnp.broadcast_to(ab_ref[...], (4, 526)).astype(jnp.float32)
        sc = s_ref[bb, 0:128, :]
        sl = s_ref[bb, 128:256, :]
        prev = None
        for di in range(5, -1, -1):
            for dj in range(5, -1, -1):
                off = di * 24 + dj
                if prev is not None:
                    acc_c = jnp.roll(acc_c, off - prev, axis=1)
                    acc_l = jnp.roll(acc_l, off - prev, axis=1)
                tap = di * 6 + dj
                acc_c = acc_c + jnp.dot(k_ref[bb, tap, 0:2, :], sc,
                                        preferred_element_type=jnp.float32)
                acc_l = acc_l + jnp.dot(k_ref[bb, tap, 2:6, :], sl,
                                        preferred_element_type=jnp.float32)
                prev = off
        out_ref[bb, 0:2, :] = acc_c[:, 0:401]
        out_ref[bb, 2:6, :] = acc_l[:, 0:401]


def kernel(z, x, tc_w, tc_b, tl_w, tl_b, sc_w, sc_b, sl_w, sl_b, adj_w, adj_b):
    f32 = jnp.float32
    bf16 = jnp.bfloat16
    B, C = z.shape[0], z.shape[1]

    zt = jnp.pad(z.astype(f32).reshape(B, C, 64).transpose(0, 2, 1),
                 ((0, 0), (0, 8), (0, 0)))
    xf = x.astype(f32).reshape(B, C, 576)
    aw = adj_w[:, :, 0, 0].astype(f32)
    wtl = jnp.einsum('pn,ncr->pcr', aw,
                     tl_w.astype(f32).reshape(4, C, C * 9)).reshape(4 * C, C * 9)
    wt_all = jnp.concatenate([tc_w.astype(f32).reshape(2 * C, C * 9), wtl], 0)
    wtT9 = wt_all.reshape(6 * C, C, 9).transpose(2, 1, 0)
    btl = (aw @ tl_b.astype(f32).reshape(4, C)).reshape(4 * C)
    btT = jnp.concatenate([tc_b.astype(f32), btl]).reshape(1, 6 * C)
    ws_all = jnp.concatenate([sc_w, sl_w], 0).astype(f32)
    ws9 = ws_all.transpose(2, 3, 0, 1).reshape(9, 2 * C, C)
    bs = jnp.concatenate([sc_b, sl_b]).astype(f32).reshape(2 * C, 1)

    tT, s_out = pl.pallas_call(
        _conv_kernel,
        out_shape=(jax.ShapeDtypeStruct((B, 48, 6 * C), bf16),
                   jax.ShapeDtypeStruct((B, 2 * C, 526), bf16)),
        grid=(B // _GB,),
        in_specs=[
            pl.BlockSpec((_GB, 72, C), lambda b: (b, 0, 0)),
            pl.BlockSpec((_GB, C, 576), lambda b: (b, 0, 0)),
            pl.BlockSpec((9, C, 6 * C), lambda b: (0, 0, 0)),
            pl.BlockSpec((1, 6 * C), lambda b: (0, 0)),
            pl.BlockSpec((9, 2 * C, C), lambda b: (0, 0, 0)),
            pl.BlockSpec((2 * C, 1), lambda b: (0, 0)),
        ],
        out_specs=(pl.BlockSpec((_GB, 48, 6 * C), lambda b: (b, 0, 0)),
                   pl.BlockSpec((_GB, 2 * C, 526), lambda b: (b, 0, 0))),
        compiler_params=_PARALLEL,
    )(zt, xf, wtT9, btT, ws9, bs)

    k_all = (tT.reshape(B, 6, 8, 6, C)[:, :, :6]
             .reshape(B, 36, 6, C))
    ab = adj_b.astype(f32).reshape(4, 1)

    out = pl.pallas_call(
        _xcorr_kernel,
        out_shape=jax.ShapeDtypeStruct((B, 8, 401), f32),
        grid=(B // _GB,),
        in_specs=[
            pl.BlockSpec((_GB, 36, 6, C), lambda b: (b, 0, 0, 0)),
            pl.BlockSpec((_GB, 2 * C, 526), lambda b: (b, 0, 0)),
            pl.BlockSpec((4, 1), lambda b: (0, 0)),
        ],
        out_specs=pl.BlockSpec((_GB, 8, 401), lambda b: (b, 0, 0)),
        compiler_params=_PARALLEL,
    )(k_all, s_out, ab)

    r = jnp.pad(out, ((0, 0), (0, 0), (0, 7))).reshape(B, 8, 17, 24)[:, :, :, :17]
    return r[:, 0:2], r[:, 2:6]


# speedup vs baseline: 12.6312x; 1.0373x over previous
import jax
import jax.numpy as jnp
from jax.experimental import pallas as pl
from jax.experimental.pallas import tpu as pltpu

_PARALLEL = pltpu.CompilerParams(dimension_semantics=("parallel",))
_GB = 8


def _conv_kernel(zt_ref, xf_ref, wtT_ref, btT_ref, ws_ref, bs_ref,
                 tT_ref, s_ref):
    for bb in range(_GB):
        zb = zt_ref[bb].astype(jnp.bfloat16)
        xb = xf_ref[bb].astype(jnp.bfloat16)
        acc_t = jnp.zeros((48, 768), jnp.float32)
        acc_s = jnp.zeros((256, 526), jnp.float32)
        for i in range(3):
            for j in range(3):
                tap = i * 3 + j
                zo = i * 8 + j
                xo = i * 24 + j
                acc_t += jnp.dot(zb[zo:zo + 48, :], wtT_ref[tap],
                                 preferred_element_type=jnp.float32)
                acc_s += jnp.dot(ws_ref[tap], xb[:, xo:xo + 526],
                                 preferred_element_type=jnp.float32)
        tT_ref[bb] = (acc_t + btT_ref[...]).astype(jnp.bfloat16)
        s_ref[bb] = (acc_s + bs_ref[...]).astype(jnp.bfloat16)


def _xcorr_kernel(k_ref, s_ref, ab_ref, out_ref):
    for bb in range(_GB):
        acc_c = jnp.zeros((2, 526), jnp.float32)
        acc_l = jnp.broadcast_to(ab_ref[...], (4, 526)).astype(jnp.float32)
        sc = s_ref[bb, 0:128, :]
        sl = s_ref[bb, 128:256, :]
        prev = None
        for di in range(5, -1, -1):
            for dj in range(5, -1, -1):
                off = di * 24 + dj
                if prev is not None:
                    acc_c = jnp.roll(acc_c, off - prev, axis=1)
                    acc_l = jnp.roll(acc_l, off - prev, axis=1)
                tap = di * 6 + dj
                acc_c = acc_c + jnp.dot(k_ref[bb, tap, 0:2, :], sc,
                                        preferred_element_type=jnp.float32)
                acc_l = acc_l + jnp.dot(k_ref[bb, tap, 2:6, :], sl,
                                        preferred_element_type=jnp.float32)
                prev = off
        out_ref[bb, 0:2, :] = acc_c[:, 0:401]
        out_ref[bb, 2:6, :] = acc_l[:, 0:401]


def kernel(z, x, tc_w, tc_b, tl_w, tl_b, sc_w, sc_b, sl_w, sl_b, adj_w, adj_b):
    f32 = jnp.float32
    bf16 = jnp.bfloat16
    B, C = z.shape[0], z.shape[1]

    zt = jnp.pad(z.astype(f32).reshape(B, C, 64).transpose(0, 2, 1),
                 ((0, 0), (0, 8), (0, 0)))
    xf = x.astype(f32).reshape(B, C, 576)
    aw = adj_w[:, :, 0, 0].astype(f32)
    wtl = jnp.einsum('pn,ncr->pcr', aw,
                     tl_w.astype(f32).reshape(4, C, C * 9)).reshape(4 * C, C * 9)
    wt_all = jnp.concatenate([tc_w.astype(f32).reshape(2 * C, C * 9), wtl], 0)
    wtT9 = wt_all.reshape(6 * C, C, 9).transpose(2, 1, 0).astype(bf16)
    btl = (aw @ tl_b.astype(f32).reshape(4, C)).reshape(4 * C)
    btT = jnp.concatenate([tc_b.astype(f32), btl]).reshape(1, 6 * C)
    ws_all = jnp.concatenate([sc_w, sl_w], 0).astype(f32)
    ws9 = ws_all.transpose(2, 3, 0, 1).reshape(9, 2 * C, C).astype(bf16)
    bs = jnp.concatenate([sc_b, sl_b]).astype(f32).reshape(2 * C, 1)

    tT, s_out = pl.pallas_call(
        _conv_kernel,
        out_shape=(jax.ShapeDtypeStruct((B, 48, 6 * C), bf16),
                   jax.ShapeDtypeStruct((B, 2 * C, 526), bf16)),
        grid=(B // _GB,),
        in_specs=[
            pl.BlockSpec((_GB, 72, C), lambda b: (b, 0, 0)),
            pl.BlockSpec((_GB, C, 576), lambda b: (b, 0, 0)),
            pl.BlockSpec((9, C, 6 * C), lambda b: (0, 0, 0)),
            pl.BlockSpec((1, 6 * C), lambda b: (0, 0)),
            pl.BlockSpec((9, 2 * C, C), lambda b: (0, 0, 0)),
            pl.BlockSpec((2 * C, 1), lambda b: (0, 0)),
        ],
        out_specs=(pl.BlockSpec((_GB, 48, 6 * C), lambda b: (b, 0, 0)),
                   pl.BlockSpec((_GB, 2 * C, 526), lambda b: (b, 0, 0))),
        compiler_params=_PARALLEL,
    )(zt, xf, wtT9, btT, ws9, bs)

    k_all = (tT.reshape(B, 6, 8, 6, C)[:, :, :6]
             .reshape(B, 36, 6, C))
    ab = adj_b.astype(f32).reshape(4, 1)

    out = pl.pallas_call(
        _xcorr_kernel,
        out_shape=jax.ShapeDtypeStruct((B, 8, 401), f32),
        grid=(B // _GB,),
        in_specs=[
            pl.BlockSpec((_GB, 36, 6, C), lambda b: (b, 0, 0, 0)),
            pl.BlockSpec((_GB, 2 * C, 526), lambda b: (b, 0, 0)),
            pl.BlockSpec((4, 1), lambda b: (0, 0)),
        ],
        out_specs=pl.BlockSpec((_GB, 8, 401), lambda b: (b, 0, 0)),
        compiler_params=_PARALLEL,
    )(k_all, s_out, ab)

    r = jnp.pad(out, ((0, 0), (0, 0), (0, 7))).reshape(B, 8, 17, 24)[:, :, :, :17]
    return r[:, 0:2], r[:, 2:6]
